# baseline (speedup 1.0000x reference)
import jax
import jax.numpy as jnp
from jax.experimental import pallas as pl
from jax.experimental.pallas import tpu as pltpu

_TEMP_INV = 10.0
_N = 2048
_R = 8
_BI = 256
_NK = _N // _BI
_C = 0.5 + _N / 2.0


def _body(pb_ref, tb_ref, p_ref, t_ref, out_ref, acc_ref):
    r = pl.program_id(0)
    k = pl.program_id(1)

    @pl.when(jnp.logical_and(r == 0, k == 0))
    def _():
        acc_ref[3] = 0.0

    @pl.when(k == 0)
    def _():
        acc_ref[0] = 0.0
        acc_ref[1] = 0.0
        acc_ref[2] = 0.0

    p_blk = pb_ref[0, 0, :].reshape(_BI, 1)
    t_blk = tb_ref[0, 0, :].reshape(_BI, 1)
    p_all = p_ref[0, 0, :].reshape(1, _N)
    t_all = t_ref[0, 0, :].reshape(1, _N)

    zp = (p_blk - p_all) * _TEMP_INV
    sp = 1.0 / (1.0 + jnp.exp(-zp))
    rp = jnp.sum(sp, axis=1) + 0.5

    zt = (t_blk - t_all) * _TEMP_INV
    st = 1.0 / (1.0 + jnp.exp(-zt))
    rt = jnp.sum(st, axis=1) + 0.5

    xb = rp - _C
    yb = rt - _C
    acc_ref[0] += jnp.sum(xb * yb)
    acc_ref[1] += jnp.sum(xb * xb)
    acc_ref[2] += jnp.sum(yb * yb)

    @pl.when(k == _NK - 1)
    def _():
        sxy = acc_ref[0] / _N
        sxx = acc_ref[1] / _N
        syy = acc_ref[2] / _N
        vx = jnp.sqrt(sxx + 1e-8)
        vy = jnp.sqrt(syy + 1e-8)
        rho = sxy / (vx * vy + 1e-8)
        acc_ref[3] += (1.0 - rho) / _R

    @pl.when(jnp.logical_and(r == _R - 1, k == _NK - 1))
    def _():
        out_ref[0, 0] = acc_ref[3]


def kernel(preds, targets):
    p3 = preds.reshape(_R, 1, _N)
    t3 = targets.reshape(_R, 1, _N)
    out = pl.pallas_call(
        _body,
        grid=(_R, _NK),
        in_specs=[
            pl.BlockSpec((1, 1, _BI), lambda r, k: (r, 0, k)),
            pl.BlockSpec((1, 1, _BI), lambda r, k: (r, 0, k)),
            pl.BlockSpec((1, 1, _N), lambda r, k: (r, 0, 0)),
            pl.BlockSpec((1, 1, _N), lambda r, k: (r, 0, 0)),
        ],
        out_specs=pl.BlockSpec(memory_space=pltpu.SMEM),
        out_shape=jax.ShapeDtypeStruct((1, 1), jnp.float32),
        scratch_shapes=[pltpu.SMEM((4,), jnp.float32)],
    )(p3, t3, p3, t3)
    return out[0, 0]


# speedup vs baseline: 1.5489x; 1.5489x over previous
import jax
import jax.numpy as jnp
from jax.experimental import pallas as pl
from jax.experimental.pallas import tpu as pltpu

_TEMP_INV = 10.0
_N = 2048
_R = 8
_BI = 256
_NK = _N // _BI
_C = 0.5 + _N / 2.0


def _body(pb_ref, tb_ref, p_ref, t_ref, out_ref, acc_ref):
    r = pl.program_id(0)
    k = pl.program_id(1)

    @pl.when(jnp.logical_and(r == 0, k == 0))
    def _():
        acc_ref[3] = 0.0

    @pl.when(k == 0)
    def _():
        acc_ref[0] = 0.0
        acc_ref[1] = 0.0
        acc_ref[2] = 0.0

    p_blk = pb_ref[0, 0, :].reshape(_BI, 1)
    t_blk = tb_ref[0, 0, :].reshape(_BI, 1)
    p_all = p_ref[0, 0, :].reshape(1, _N)
    t_all = t_ref[0, 0, :].reshape(1, _N)

    xb = 0.5 * jnp.sum(jnp.tanh((p_blk - p_all) * (0.5 * _TEMP_INV)), axis=1)
    yb = 0.5 * jnp.sum(jnp.tanh((t_blk - t_all) * (0.5 * _TEMP_INV)), axis=1)
    acc_ref[0] += jnp.sum(xb * yb)
    acc_ref[1] += jnp.sum(xb * xb)
    acc_ref[2] += jnp.sum(yb * yb)

    @pl.when(k == _NK - 1)
    def _():
        sxy = acc_ref[0] / _N
        sxx = acc_ref[1] / _N
        syy = acc_ref[2] / _N
        vx = jnp.sqrt(sxx + 1e-8)
        vy = jnp.sqrt(syy + 1e-8)
        rho = sxy / (vx * vy + 1e-8)
        acc_ref[3] += (1.0 - rho) / _R

    @pl.when(jnp.logical_and(r == _R - 1, k == _NK - 1))
    def _():
        out_ref[0, 0] = acc_ref[3]


def kernel(preds, targets):
    p3 = preds.reshape(_R, 1, _N)
    t3 = targets.reshape(_R, 1, _N)
    out = pl.pallas_call(
        _body,
        grid=(_R, _NK),
        in_specs=[
            pl.BlockSpec((1, 1, _BI), lambda r, k: (r, 0, k)),
            pl.BlockSpec((1, 1, _BI), lambda r, k: (r, 0, k)),
            pl.BlockSpec((1, 1, _N), lambda r, k: (r, 0, 0)),
            pl.BlockSpec((1, 1, _N), lambda r, k: (r, 0, 0)),
        ],
        out_specs=pl.BlockSpec(memory_space=pltpu.SMEM),
        out_shape=jax.ShapeDtypeStruct((1, 1), jnp.float32),
        scratch_shapes=[pltpu.SMEM((4,), jnp.float32)],
    )(p3, t3, p3, t3)
    return out[0, 0]
